# Initial kernel scaffold; baseline (speedup 1.0000x reference)
import jax
import jax.numpy as jnp
from jax.experimental import pallas as pl


def kernel(x, W_enc, b_enc, W_dec, b_dec, mean):
    raise NotImplementedError("write your pallas kernel here")


# speedup vs baseline: 9.0260x; 9.0260x over previous
import jax
import jax.numpy as jnp
from jax.experimental import pallas as pl

INPUT_DIM = 768
HIDDEN_DIM = 16384
K = 64
N_TOKENS = 4096
BLOCK_A = 32
BLOCK_B = 128


def _key_to_float(k):
    b = k ^ ((k >> 31) & jnp.int32(0x7FFFFFFF))
    return jax.lax.bitcast_convert_type(b, jnp.float32)


def _enc_body(x_ref, mean_ref, we_ref, be_ref, z_ref, thr_ref):
    xc = x_ref[...] - mean_ref[...]
    z = jax.lax.dot_general(
        xc, we_ref[...], (((1,), (0,)), ((), ())),
        preferred_element_type=jnp.float32,
    )
    z = z + be_ref[...]
    z_ref[...] = z

    lo0 = jnp.full((BLOCK_A, 1), jnp.iinfo(jnp.int32).min, jnp.int32)
    hi0 = jnp.full((BLOCK_A, 1), jnp.iinfo(jnp.int32).max, jnp.int32)

    def step(_, lohi):
        lo, hi = lohi
        mid = (lo >> 1) + (hi >> 1) + (lo & hi & 1)
        fmid = _key_to_float(mid)
        cnt = jnp.sum((z > fmid).astype(jnp.float32), axis=1, keepdims=True)
        pred = cnt >= K
        return jnp.where(pred, mid, lo), jnp.where(pred, hi, mid)

    lo, _ = jax.lax.fori_loop(0, 32, step, (lo0, hi0))
    thr_ref[...] = _key_to_float(lo)


def _dec_body(z_ref, thr_ref, wd_ref, bd_ref, mean_ref, o_ref):
    z = z_ref[...]
    sparse = jnp.where(z > thr_ref[...], z, 0.0).astype(jnp.bfloat16)
    dec = jax.lax.dot_general(
        sparse, wd_ref[...], (((1,), (0,)), ((), ())),
        preferred_element_type=jnp.float32,
    )
    o_ref[...] = dec + bd_ref[...] + mean_ref[...]


@jax.jit
def kernel(x, W_enc, b_enc, W_dec, b_dec, mean):
    mean2 = mean.reshape(1, INPUT_DIM)
    z, thr = pl.pallas_call(
        _enc_body,
        grid=(N_TOKENS // BLOCK_A,),
        in_specs=[
            pl.BlockSpec((BLOCK_A, INPUT_DIM), lambda i: (i, 0)),
            pl.BlockSpec((1, INPUT_DIM), lambda i: (0, 0)),
            pl.BlockSpec((INPUT_DIM, HIDDEN_DIM), lambda i: (0, 0)),
            pl.BlockSpec((1, HIDDEN_DIM), lambda i: (0, 0)),
        ],
        out_specs=[
            pl.BlockSpec((BLOCK_A, HIDDEN_DIM), lambda i: (i, 0)),
            pl.BlockSpec((BLOCK_A, 1), lambda i: (i, 0)),
        ],
        out_shape=[
            jax.ShapeDtypeStruct((N_TOKENS, HIDDEN_DIM), jnp.float32),
            jax.ShapeDtypeStruct((N_TOKENS, 1), jnp.float32),
        ],
    )(x, mean2, W_enc, b_enc.reshape(1, HIDDEN_DIM))

    return pl.pallas_call(
        _dec_body,
        grid=(N_TOKENS // BLOCK_B,),
        in_specs=[
            pl.BlockSpec((BLOCK_B, HIDDEN_DIM), lambda i: (i, 0)),
            pl.BlockSpec((BLOCK_B, 1), lambda i: (i, 0)),
            pl.BlockSpec((HIDDEN_DIM, INPUT_DIM), lambda i: (0, 0)),
            pl.BlockSpec((1, INPUT_DIM), lambda i: (0, 0)),
            pl.BlockSpec((1, INPUT_DIM), lambda i: (0, 0)),
        ],
        out_specs=pl.BlockSpec((BLOCK_B, INPUT_DIM), lambda i: (i, 0)),
        out_shape=jax.ShapeDtypeStruct((N_TOKENS, INPUT_DIM), jnp.float32),
    )(z, thr, W_dec.astype(jnp.bfloat16), b_dec.reshape(1, INPUT_DIM), mean2)
